# baseline (speedup 1.0000x reference)
import jax
import jax.numpy as jnp
from jax import lax
from jax.experimental import pallas as pl
from jax.experimental.pallas import tpu as pltpu
from jax.experimental.pallas import tpu_sc as plsc

VOCAB = 1000000
D = 64
D2 = 128
ROWS = 4096
COLS = 200
B_TOTAL = ROWS * COLS
NC = 2
NS = 16
NW = NC * NS
PER_W = B_TOTAL // NW
STREAM = 128
BUF = 256
SPB = BUF // STREAM
NSTEP = PER_W // BUF
LANES = 16
VPR = D // LANES
RU = 8
SCALE = 8.0


def _body(p_hbm, off_hbm, tab_hbm, out_hbm, idx_v, gb0, gb1, sb0, sb1,
          of0, of1, gsem0, gsem1, ssem0, ssem1, osem0, osem1):
  c = lax.axis_index("c")
  s = lax.axis_index("s")
  wid = s * NC + c
  base = wid * PER_W
  base2 = wid * (PER_W // 2)

  gbufs = (gb0, gb1)
  sbufs = (sb0, sb1)
  obufs = (of0, of1)
  gsems = (gsem0, gsem1)
  ssems = (ssem0, ssem1)
  osems = (osem0, osem1)

  pltpu.sync_copy(p_hbm.at[pl.ds(base, PER_W)], idx_v)

  def start_gather(j, b):
    pltpu.async_copy(off_hbm.at[pl.ds(base + j * BUF, BUF)], obufs[b],
                     osems[b])
    for q in range(SPB):
      pltpu.async_copy(
          tab_hbm.at[idx_v.at[pl.ds(j * BUF + q * STREAM, STREAM)]],
          gbufs[b].at[pl.ds(q * STREAM, STREAM)],
          gsems[b],
      )

  def wait_gather(j, b):
    pltpu.make_async_copy(off_hbm.at[pl.ds(base + j * BUF, BUF)], obufs[b],
                          osems[b]).wait()
    for q in range(SPB):
      pltpu.make_async_copy(
          tab_hbm.at[idx_v.at[pl.ds(j * BUF + q * STREAM, STREAM)]],
          gbufs[b].at[pl.ds(q * STREAM, STREAM)],
          gsems[b],
      ).wait()

  def start_scatter(j, b):
    pltpu.async_copy(
        sbufs[b], out_hbm.at[pl.ds(base2 + j * (BUF // 2), BUF // 2)],
        ssems[b])

  def wait_scatter(j, b):
    pltpu.make_async_copy(
        sbufs[b], out_hbm.at[pl.ds(base2 + j * (BUF // 2), BUF // 2)],
        ssems[b]).wait()

  start_gather(0, 0)
  start_gather(1, 1)

  def outer(jj, carry):
    for b in range(2):
      j = 2 * jj + b
      wait_gather(j, b)

      @pl.when(j >= 2)
      def _():
        wait_scatter(j - 2, b)

      gb, sb, ob = gbufs[b], sbufs[b], obufs[b]

      @plsc.parallel_loop(0, BUF, step=LANES)
      def scale8(i):
        ov = ob[pl.ds(i, LANES)]
        for r in range(LANES):
          o = ov[r]
          half = (r % 2) * D
          for k in range(VPR):
            sb[(i + r) // 2, pl.ds(half + k * LANES, LANES)] = (
                gb[i + r, pl.ds(o + k * LANES, LANES)] * SCALE
            )

      @pl.when(j + 2 < NSTEP)
      def _():
        start_gather(j + 2, b)

      start_scatter(j, b)
    return carry

  lax.fori_loop(0, NSTEP // 2, outer, 0)

  wait_scatter(NSTEP - 2, 0)
  wait_scatter(NSTEP - 1, 1)


@jax.jit
def _embed(p, off, tab2):
  mesh = plsc.VectorSubcoreMesh(core_axis_name="c", subcore_axis_name="s")
  kfn = pl.kernel(
      _body,
      out_type=jax.ShapeDtypeStruct((B_TOTAL // 2, D2), jnp.float32),
      mesh=mesh,
      scratch_types=[
          pltpu.VMEM((PER_W,), jnp.int32),
          pltpu.VMEM((BUF, D2), jnp.float32),
          pltpu.VMEM((BUF, D2), jnp.float32),
          pltpu.VMEM((BUF // 2, D2), jnp.float32),
          pltpu.VMEM((BUF // 2, D2), jnp.float32),
          pltpu.VMEM((BUF,), jnp.int32),
          pltpu.VMEM((BUF,), jnp.int32),
          pltpu.SemaphoreType.DMA,
          pltpu.SemaphoreType.DMA,
          pltpu.SemaphoreType.DMA,
          pltpu.SemaphoreType.DMA,
          pltpu.SemaphoreType.DMA,
          pltpu.SemaphoreType.DMA,
      ],
  )
  return kfn(p, off, tab2)


def kernel(x, input_embedding):
  x_flat = x.reshape(-1).astype(jnp.int32)
  p = x_flat >> 1
  off = (x_flat & 1) << 6
  tab2 = input_embedding.reshape(VOCAB // 2, D2)
  out = _embed(p, off, tab2)
  return out.reshape(ROWS, COLS, D)


# speedup vs baseline: 1.0911x; 1.0911x over previous
import jax
import jax.numpy as jnp
from jax import lax
from jax.experimental import pallas as pl
from jax.experimental.pallas import tpu as pltpu
from jax.experimental.pallas import tpu_sc as plsc

VOCAB = 1000000
D = 64
ROWS = 4096
COLS = 200
B_TOTAL = ROWS * COLS
NC = 2
NS = 16
NW = NC * NS
XROWS_W = ROWS // NW
PER_W = B_TOTAL // NW
BUF = COLS
NSTEP = XROWS_W
STREAMS = (128, 72)
LANES = 16
VPR = D // LANES
RU = 8
SCALE = 8.0


def _body(x_hbm, tab_hbm, out_hbm, idx_v, gb0, gb1, sb0, sb1,
          gsem0, gsem1, ssem0, ssem1):
  c = lax.axis_index("c")
  s = lax.axis_index("s")
  wid = s * NC + c
  base = wid * PER_W
  xbase = wid * XROWS_W

  gbufs = (gb0, gb1)
  sbufs = (sb0, sb1)
  gsems = (gsem0, gsem1)
  ssems = (ssem0, ssem1)

  pltpu.sync_copy(x_hbm.at[pl.ds(base, PER_W)], idx_v)

  def start_gather(j, b):
    off = 0
    for n in STREAMS:
      pltpu.async_copy(
          tab_hbm.at[idx_v.at[pl.ds(j * BUF + off, n)]],
          gbufs[b].at[pl.ds(off, n)],
          gsems[b],
      )
      off += n

  def wait_gather(j, b):
    off = 0
    for n in STREAMS:
      pltpu.make_async_copy(
          tab_hbm.at[idx_v.at[pl.ds(j * BUF + off, n)]],
          gbufs[b].at[pl.ds(off, n)],
          gsems[b],
      ).wait()
      off += n

  def start_scatter(j, b):
    pltpu.async_copy(sbufs[b], out_hbm.at[pl.ds(xbase + j, 1)], ssems[b])

  def wait_scatter(j, b):
    pltpu.make_async_copy(sbufs[b], out_hbm.at[pl.ds(xbase + j, 1)],
                          ssems[b]).wait()

  start_gather(0, 0)
  start_gather(1, 1)

  def outer(jj, carry):
    for b in range(2):
      j = 2 * jj + b
      wait_gather(j, b)

      @pl.when(j >= 2)
      def _():
        wait_scatter(j - 2, b)

      gb, sb = gbufs[b], sbufs[b]

      @plsc.parallel_loop(0, BUF, step=RU)
      def scale8(i):
        for r in range(RU):
          for k in range(VPR):
            sb[0, i + r, pl.ds(k * LANES, LANES)] = (
                gb[i + r, pl.ds(k * LANES, LANES)] * SCALE
            )

      @pl.when(j + 2 < NSTEP)
      def _():
        start_gather(j + 2, b)

      start_scatter(j, b)
    return carry

  lax.fori_loop(0, NSTEP // 2, outer, 0)

  wait_scatter(NSTEP - 2, 0)
  wait_scatter(NSTEP - 1, 1)


@jax.jit
def _embed(x_flat, table):
  mesh = plsc.VectorSubcoreMesh(core_axis_name="c", subcore_axis_name="s")
  kfn = pl.kernel(
      _body,
      out_type=jax.ShapeDtypeStruct((ROWS, COLS, D), jnp.float32),
      mesh=mesh,
      scratch_types=[
          pltpu.VMEM((PER_W,), jnp.int32),
          pltpu.VMEM((BUF, D), jnp.float32),
          pltpu.VMEM((BUF, D), jnp.float32),
          pltpu.VMEM((1, BUF, D), jnp.float32),
          pltpu.VMEM((1, BUF, D), jnp.float32),
          pltpu.SemaphoreType.DMA,
          pltpu.SemaphoreType.DMA,
          pltpu.SemaphoreType.DMA,
          pltpu.SemaphoreType.DMA,
      ],
      compiler_params=pltpu.CompilerParams(use_tc_tiling_on_sc=False),
  )
  return kfn(x_flat, table)


def kernel(x, input_embedding):
  x_flat = x.reshape(-1).astype(jnp.int32)
  return _embed(x_flat, input_embedding)


# speedup vs baseline: 1.3342x; 1.2228x over previous
import jax
import jax.numpy as jnp
from jax import lax
from jax.experimental import pallas as pl
from jax.experimental.pallas import tpu as pltpu
from jax.experimental.pallas import tpu_sc as plsc

VOCAB = 1000000
D = 64
ROWS = 4096
COLS = 200
B_TOTAL = ROWS * COLS
NC = 2
NS = 16
NW = NC * NS
XROWS_W = ROWS // NW
PER_W = B_TOTAL // NW
BUF = COLS
NSTEP = XROWS_W
STREAMS = (128, 72)
LANES = 16
VPR = D // LANES
RU = 8
SCALE = 8.0


def _body(x_hbm, tab_hbm, out_hbm, idx_v, gb0, gb1, sb0, sb1,
          gsem0, gsem1, ssem0, ssem1):
  c = lax.axis_index("c")
  s = lax.axis_index("s")
  wid = s * NC + c
  base = wid * PER_W
  xbase = wid * XROWS_W

  gbufs = (gb0, gb1)
  sbufs = (sb0, sb1)
  gsems = (gsem0, gsem1)
  ssems = (ssem0, ssem1)

  pltpu.sync_copy(x_hbm.at[pl.ds(base, PER_W)], idx_v)

  def start_gather(j, b):
    off = 0
    for n in STREAMS:
      pltpu.async_copy(
          tab_hbm.at[idx_v.at[pl.ds(j * BUF + off, n)]],
          gbufs[b].at[pl.ds(off, n)],
          gsems[b],
      )
      off += n

  def wait_gather(j, b):
    off = 0
    for n in STREAMS:
      pltpu.make_async_copy(
          tab_hbm.at[idx_v.at[pl.ds(j * BUF + off, n)]],
          gbufs[b].at[pl.ds(off, n)],
          gsems[b],
      ).wait()
      off += n

  def start_scatter(j, b):
    pltpu.async_copy(sbufs[b], out_hbm.at[pl.ds(xbase + j, 1)], ssems[b])

  def wait_scatter(j, b):
    pltpu.make_async_copy(sbufs[b], out_hbm.at[pl.ds(xbase + j, 1)],
                          ssems[b]).wait()

  start_gather(0, 0)
  start_gather(1, 1)

  def outer(jj, carry):
    for b in range(2):
      j = 2 * jj + b
      wait_gather(j, b)

      @pl.when(j >= 2)
      def _():
        wait_scatter(j - 2, b)

      gb, sb = gbufs[b], sbufs[b]

      @plsc.parallel_loop(0, BUF, step=RU)
      def scale8(i):
        for r in range(RU):
          for k in range(VPR):
            sb[0, i + r, pl.ds(k * LANES, LANES)] = (
                gb[i + r, pl.ds(k * LANES, LANES)] * SCALE
            )

      @pl.when(j + 2 < NSTEP)
      def _():
        start_gather(j + 2, b)

      start_scatter(j, b)
    return carry

  lax.fori_loop(0, NSTEP // 2, outer, 0)

  wait_scatter(NSTEP - 2, 0)
  wait_scatter(NSTEP - 1, 1)


@jax.jit
def _embed(x_flat, table):
  mesh = plsc.VectorSubcoreMesh(core_axis_name="c", subcore_axis_name="s")
  kfn = pl.kernel(
      _body,
      out_type=jax.ShapeDtypeStruct((ROWS, COLS, 2 * D), jnp.float32),
      mesh=mesh,
      scratch_types=[
          pltpu.VMEM((PER_W,), jnp.int32),
          pltpu.VMEM((BUF, D), jnp.float32),
          pltpu.VMEM((BUF, D), jnp.float32),
          pltpu.VMEM((1, BUF, 2 * D), jnp.float32),
          pltpu.VMEM((1, BUF, 2 * D), jnp.float32),
          pltpu.SemaphoreType.DMA,
          pltpu.SemaphoreType.DMA,
          pltpu.SemaphoreType.DMA,
          pltpu.SemaphoreType.DMA,
      ],
      compiler_params=pltpu.CompilerParams(use_tc_tiling_on_sc=False),
  )
  return kfn(x_flat, table)


def kernel(x, input_embedding):
  x_flat = x.reshape(-1).astype(jnp.int32)
  out = _embed(x_flat, input_embedding)
  return out[:, :, :D]


# speedup vs baseline: 1.3343x; 1.0001x over previous
import jax
import jax.numpy as jnp
from jax import lax
from jax.experimental import pallas as pl
from jax.experimental.pallas import tpu as pltpu
from jax.experimental.pallas import tpu_sc as plsc

VOCAB = 1000000
D = 64
ROWS = 4096
COLS = 200
B_TOTAL = ROWS * COLS
NC = 2
NS = 16
NW = NC * NS
XROWS_W = ROWS // NW
PER_W = B_TOTAL // NW
BUF = COLS
NSTEP = XROWS_W
STREAMS = (128, 72)
LANES = 16
VPR = D // LANES
RU = 8
SCALE = 8.0


def _body(x_hbm, tab_hbm, out_hbm, idx_v, gb0, gb1, sb0, sb1,
          gsem0, gsem1, ssem0, ssem1):
  c = lax.axis_index("c")
  s = lax.axis_index("s")
  wid = s * NC + c
  base = wid * PER_W
  xbase = wid * XROWS_W

  gbufs = (gb0, gb1)
  sbufs = (sb0, sb1)
  gsems = (gsem0, gsem1)
  ssems = (ssem0, ssem1)

  pltpu.sync_copy(x_hbm.at[pl.ds(base, PER_W)], idx_v)

  def start_gather(j, b):
    off = 0
    for n in STREAMS:
      pltpu.async_copy(
          tab_hbm.at[idx_v.at[pl.ds(j * BUF + off, n)]],
          gbufs[b].at[pl.ds(off, n)],
          gsems[b],
      )
      off += n

  def wait_gather(j, b):
    off = 0
    for n in STREAMS:
      pltpu.make_async_copy(
          tab_hbm.at[idx_v.at[pl.ds(j * BUF + off, n)]],
          gbufs[b].at[pl.ds(off, n)],
          gsems[b],
      ).wait()
      off += n

  def start_scatter(j, b):
    pltpu.async_copy(sbufs[b], out_hbm.at[pl.ds(base + j * BUF, BUF)],
                     ssems[b])

  def wait_scatter(j, b):
    pltpu.make_async_copy(sbufs[b], out_hbm.at[pl.ds(base + j * BUF, BUF)],
                          ssems[b]).wait()

  start_gather(0, 0)
  start_gather(1, 1)

  def outer(jj, carry):
    for b in range(2):
      j = 2 * jj + b
      wait_gather(j, b)

      @pl.when(j >= 2)
      def _():
        wait_scatter(j - 2, b)

      gb, sb = gbufs[b], sbufs[b]

      @plsc.parallel_loop(0, BUF, step=RU)
      def scale8(i):
        for r in range(RU):
          for k in range(VPR):
            sb[i + r, pl.ds(k * LANES, LANES)] = (
                gb[i + r, pl.ds(k * LANES, LANES)] * SCALE
            )

      @pl.when(j + 2 < NSTEP)
      def _():
        start_gather(j + 2, b)

      start_scatter(j, b)
    return carry

  lax.fori_loop(0, NSTEP // 2, outer, 0)

  wait_scatter(NSTEP - 2, 0)
  wait_scatter(NSTEP - 1, 1)


@jax.jit
def _embed(x_flat, table):
  mesh = plsc.VectorSubcoreMesh(core_axis_name="c", subcore_axis_name="s")
  kfn = pl.kernel(
      _body,
      out_type=jax.ShapeDtypeStruct((B_TOTAL, 2 * D), jnp.float32),
      mesh=mesh,
      scratch_types=[
          pltpu.VMEM((PER_W,), jnp.int32),
          pltpu.VMEM((BUF, D), jnp.float32),
          pltpu.VMEM((BUF, D), jnp.float32),
          pltpu.VMEM((BUF, 2 * D), jnp.float32),
          pltpu.VMEM((BUF, 2 * D), jnp.float32),
          pltpu.SemaphoreType.DMA,
          pltpu.SemaphoreType.DMA,
          pltpu.SemaphoreType.DMA,
          pltpu.SemaphoreType.DMA,
      ],
      compiler_params=pltpu.CompilerParams(use_tc_tiling_on_sc=False),
  )
  return kfn(x_flat, table)


def kernel(x, input_embedding):
  x_flat = x.reshape(-1).astype(jnp.int32)
  out = _embed(x_flat, input_embedding)
  return out[:, :D].reshape(ROWS, COLS, D)
